# baseline (speedup 1.0000x reference)
import functools

import jax
import jax.numpy as jnp
from jax import lax
from jax.experimental import pallas as pl
from jax.experimental.pallas import tpu as pltpu
from jax.experimental.pallas import tpu_sc as plsc

N = 10000
D = 128
H = 8
DH = 16
L = 2
NPAD = 10240
NC, NS, LANES = 2, 16, 16
NW = NC * NS
C = 32
AW = 144
RPT = NPAD // NS

_ET = [('functions', 'functions'), ('classes', 'functions'),
       ('functions', 'classes'), ('code', 'functions')]


def _mm_body(x_ref, w_ref, o_ref):
    o_ref[...] = jnp.dot(x_ref[...], w_ref[...],
                         preferred_element_type=jnp.float32)


def _mm(x, w, bm=1024):
    m, k = x.shape
    n = w.shape[1]
    return pl.pallas_call(
        _mm_body,
        grid=(m // bm,),
        in_specs=[pl.BlockSpec((bm, k), lambda i: (i, 0)),
                  pl.BlockSpec((k, n), lambda i: (0, 0))],
        out_specs=pl.BlockSpec((bm, n), lambda i: (i, 0)),
        out_shape=jax.ShapeDtypeStruct((m, n), jnp.float32),
    )(x, w)


def _bmm_body(a_ref, b_ref, o_ref):
    o_ref[0] = jnp.dot(a_ref[0], b_ref[0], preferred_element_type=jnp.float32)


def _bmm(a, b):
    bsz = a.shape[0]
    return pl.pallas_call(
        _bmm_body,
        grid=(bsz,),
        in_specs=[pl.BlockSpec((1, D, D), lambda i: (i, 0, 0)),
                  pl.BlockSpec((1, D, D), lambda i: (i, 0, 0))],
        out_specs=pl.BlockSpec((1, D, D), lambda i: (i, 0, 0)),
        out_shape=jax.ShapeDtypeStruct((bsz, D, D), jnp.float32),
    )(a, b)


def _combine_body(n_et, s_ref, d_ref, x_ref, wa_ref, o_ref):
    m = None
    for i in range(n_et):
        num = s_ref[2 * i] + s_ref[2 * i + 1]
        den = d_ref[2 * i] + d_ref[2 * i + 1]
        den_w = jnp.concatenate([den] * DH, axis=1)
        agg = num / (den_w + 1e-9)
        m = agg if m is None else jnp.maximum(m, agg)
    g = jax.nn.gelu(m)
    o_ref[...] = jnp.maximum(
        jnp.dot(g, wa_ref[...], preferred_element_type=jnp.float32)
        + x_ref[...], 0.0)


def _combine(parts, dens, x, wa, bm=1024):
    ne2 = parts.shape[0]
    return pl.pallas_call(
        functools.partial(_combine_body, ne2 // 2),
        grid=(NPAD // bm,),
        in_specs=[pl.BlockSpec((ne2, bm, D), lambda i: (0, i, 0)),
                  pl.BlockSpec((ne2, bm, H), lambda i: (0, i, 0)),
                  pl.BlockSpec((bm, D), lambda i: (i, 0)),
                  pl.BlockSpec((D, D), lambda i: (0, 0))],
        out_specs=pl.BlockSpec((bm, D), lambda i: (i, 0)),
        out_shape=jax.ShapeDtypeStruct((NPAD, D), jnp.float32),
    )(parts, dens, x, wa)


def _mlp_body(x_ref, t_ref, w1a_ref, w1b_ref, b1_ref, w2_ref, b2_ref, o_ref):
    xb = x_ref[...]
    xn = xb / (jnp.sqrt(jnp.sum(xb * xb, axis=1, keepdims=True)) + 1e-12)
    t = t_ref[...]
    tn = t / (jnp.sqrt(jnp.sum(t * t)) + 1e-12)
    bias = b1_ref[...] + jnp.dot(tn, w1b_ref[...],
                                 preferred_element_type=jnp.float32)
    hh = jnp.maximum(
        jnp.dot(xn, w1a_ref[...], preferred_element_type=jnp.float32) + bias,
        0.0)
    o_ref[...] = jnp.dot(hh, w2_ref[...],
                         preferred_element_type=jnp.float32) + b2_ref[...]


def _mlp(x, t, w1a, w1b, b1, w2, b2, bm=1024):
    m = x.shape[0]
    return pl.pallas_call(
        _mlp_body,
        grid=(m // bm,),
        in_specs=[pl.BlockSpec((bm, D), lambda i: (i, 0)),
                  pl.BlockSpec((1, D), lambda i: (0, 0)),
                  pl.BlockSpec((D, D), lambda i: (0, 0)),
                  pl.BlockSpec((D, D), lambda i: (0, 0)),
                  pl.BlockSpec((1, D), lambda i: (0, 0)),
                  pl.BlockSpec((D, D), lambda i: (0, 0)),
                  pl.BlockSpec((1, D), lambda i: (0, 0))],
        out_specs=pl.BlockSpec((bm, D), lambda i: (i, 0)),
        out_shape=jax.ShapeDtypeStruct((m, D), jnp.float32),
    )(x, t, w1a, w1b, b1, w2, b2)


NPD = NPAD // LANES


def _perm(v, idx):
    return lax.gather(v, idx[:, None],
                      lax.GatherDimensionNumbers((), (0,), (0,)), (1,),
                      mode=lax.GatherScatterMode.PROMISE_IN_BOUNDS)


def _make_edge_kernel(epad, interpret=False):
    ew = epad // NW
    chunks = ew // C
    mesh = plsc.VectorSubcoreMesh(core_axis_name="c", subcore_axis_name="s",
                                  num_cores=NC, num_subcores=NS)

    @functools.partial(
        pl.kernel,
        out_type=(jax.ShapeDtypeStruct((NC, NPAD, D), jnp.float32),
                  jax.ShapeDtypeStruct((NC, NPD, D), jnp.float32)),
        mesh=mesh,
        interpret=interpret,
        compiler_params=pltpu.CompilerParams(needs_layout_passes=False),
        scratch_types=[
            pltpu.VMEM((C,), jnp.int32),
            pltpu.VMEM((C,), jnp.int32),
            pltpu.VMEM((C,), jnp.int32),
            pltpu.VMEM((C,), jnp.int32),
            pltpu.VMEM((C,), jnp.int32),
            pltpu.VMEM((C,), jnp.int32),
            pltpu.VMEM((C,), jnp.int32),
            pltpu.VMEM((C,), jnp.int32),
            pltpu.VMEM((C, 2 * D), jnp.float32),
            pltpu.VMEM((C, 2 * D), jnp.float32),
            pltpu.VMEM((C, D), jnp.float32),
            pltpu.VMEM((C, D), jnp.float32),
            pltpu.VMEM((C, D), jnp.float32),
            pltpu.VMEM((C, D), jnp.float32),
            pltpu.VMEM((C, LANES), jnp.float32),
            pltpu.VMEM((C, D), jnp.float32),
            pltpu.VMEM_SHARED((NPAD, D), jnp.float32),
            pltpu.VMEM_SHARED((NPD, D), jnp.float32),
            pltpu.SemaphoreType.DMA,
            pltpu.SemaphoreType.DMA,
            pltpu.SemaphoreType.DMA,
            pltpu.SemaphoreType.DMA,
            pltpu.SemaphoreType.DMA,
            pltpu.SemaphoreType.DMA,
            pltpu.SemaphoreType.DMA,
        ],
    )
    def edge_kernel(kv_hbm, q_hbm, src_hbm, dst_hbm, out_hbm, outd_hbm,
                    src0, src1, dst0, dst1, dsc0, dsc1, dhi, pcb,
                    kv0, kv1, q0, q1,
                    ob0, ob1, exbuf, dbuf, acc, accd,
                    gsem0, gsem1, isem0, isem1, ssem0, ssem1, dsem):
        cid = lax.axis_index("c")
        sid = lax.axis_index("s")
        wid = sid * NC + cid
        iot = lax.iota(jnp.int32, LANES)
        zeros16 = jnp.zeros((LANES,), jnp.float32)
        kvb = (kv0, kv1)
        qb = (q0, q1)
        srcb = (src0, src1)
        dstb = (dst0, dst1)
        gsem = (gsem0, gsem1)
        isem = (isem0, isem1)

        def ifetch(g, b):
            eb = pl.multiple_of(wid * ew + g * C, C)
            pltpu.async_copy(src_hbm.at[pl.ds(eb, C)], srcb[b], isem[b])
            pltpu.async_copy(dst_hbm.at[pl.ds(eb, C)], dstb[b], isem[b])

        def iwait(b):
            pltpu.make_async_copy(src_hbm.at[pl.ds(0, C)], srcb[b],
                                  isem[b]).wait()
            pltpu.make_async_copy(dst_hbm.at[pl.ds(0, C)], dstb[b],
                                  isem[b]).wait()

        def zrow(r, carry):
            for jj in range(D // LANES):
                ob0[r, pl.ds(jj * LANES, LANES)] = zeros16
                dbuf[r, pl.ds(jj * LANES, LANES)] = zeros16
            return carry
        lax.fori_loop(0, C, zrow, 0)
        base_r = sid * RPT
        for j in range(RPT // C):
            pltpu.sync_copy(ob0, acc.at[pl.ds(base_r + j * C, C)])
        rd = NPD // NS
        off = 0
        while off < rd:
            step = min(C, rd - off)
            pltpu.sync_copy(dbuf.at[pl.ds(0, step)],
                            accd.at[pl.ds(sid * rd + off, step)])
            off += step
        plsc.subcore_barrier()

        def gissue(b):
            pltpu.async_copy(kv_hbm.at[srcb[b]], kvb[b], gsem[b])
            pltpu.async_copy(q_hbm.at[dstb[b]], qb[b], gsem[b])

        def gwait(b):
            pltpu.make_async_copy(kv_hbm.at[pl.ds(0, C)], kvb[b],
                                  gsem[b]).wait()
            pltpu.make_async_copy(q_hbm.at[pl.ds(0, C)], qb[b],
                                  gsem[b]).wait()

        ifetch(0, 0)
        iwait(0)
        gissue(0)
        ifetch(1, 1)

        obb = (ob0, ob1)
        dscb = (dsc0, dsc1)
        ssem = (ssem0, ssem1)
        rot8 = iot ^ 8
        low8 = iot & 7

        def swait(b):
            pltpu.make_async_copy(out_hbm.at[0, pl.ds(0, C)], obb[b],
                                  ssem[b]).wait()

        def pair(p, carry):
            for b in range(2):
                g = p * 2 + b

                gwait(b)

                @pl.when(g + 1 < chunks)
                def _issue_next():
                    iwait(1 - b)
                    gissue(1 - b)

                @pl.when(g >= 2)
                def _drain_prev_scatter():
                    swait(b)

                kvbuf = kvb[b]
                qbuf = qb[b]
                obuf = obb[b]

                def edge(e, ecarry):
                    t = [qbuf[e, pl.ds(LANES * j, LANES)]
                         * kvbuf[e, pl.ds(LANES * j, LANES)]
                         for j in range(H)]
                    while len(t) > 1:
                        t = [t[2 * j] + t[2 * j + 1]
                             for j in range(len(t) // 2)]
                    sv = t[0] + _perm(t[0], rot8)
                    ev = jnp.exp(sv)
                    exbuf[e, pl.ds(0, LANES)] = ev
                    ehv = _perm(ev, low8)
                    for j in range(H):
                        obuf[e, pl.ds(LANES * j, LANES)] = (
                            kvbuf[e, pl.ds(D + LANES * j, LANES)] * ehv)
                    return ecarry
                lax.fori_loop(0, C, edge, 0, unroll=16)

                @pl.when(g >= 1)
                def _drain_den():
                    pltpu.make_async_copy(out_hbm.at[0, pl.ds(0, C)], dbuf,
                                          dsem).wait()
                    for j2 in range(C // LANES):
                        r16 = j2 * LANES + iot
                        pc = pcb[pl.ds(j2 * LANES, LANES)]
                        for h in range(H):
                            plsc.store_scatter(dbuf, [r16, pc + h], zeros16)

                for j2 in range(C // LANES):
                    r16 = j2 * LANES + iot
                    dst16 = dstb[b][pl.ds(j2 * LANES, LANES)]
                    dscb[b][pl.ds(j2 * LANES, LANES)] = dst16
                    dhi[pl.ds(j2 * LANES, LANES)] = (
                        lax.shift_right_logical(dst16, 4))
                    pcol = (dst16 & (LANES - 1)) * H
                    pcb[pl.ds(j2 * LANES, LANES)] = pcol
                    for h in range(H):
                        exh = plsc.load_gather(
                            exbuf, [r16, jnp.full((LANES,), h, jnp.int32)])
                        plsc.store_scatter(dbuf, [r16, pcol + h], exh)

                pltpu.async_copy(obuf, acc.at[dscb[b]], ssem[b], add=True)
                pltpu.async_copy(dbuf, accd.at[dhi], dsem, add=True)

                @pl.when(g + 2 < chunks)
                def _fetch_ahead():
                    ifetch(g + 2, b)
            return carry
        lax.fori_loop(0, chunks // 2, pair, 0)
        swait(0)
        swait(1)
        pltpu.make_async_copy(out_hbm.at[0, pl.ds(0, C)], dbuf, dsem).wait()
        plsc.subcore_barrier()
        for j in range(RPT // C):
            r0 = base_r + j * C
            pltpu.sync_copy(acc.at[pl.ds(r0, C)], out_hbm.at[cid, pl.ds(r0, C)])
        pltpu.sync_copy(accd.at[pl.ds(sid * rd, rd)],
                        outd_hbm.at[cid, pl.ds(sid * rd, rd)])

    return edge_kernel


_EDGE_KERNEL_CACHE = {}


def _edge(kv, q, src, dst):
    epad = src.shape[0]
    if epad not in _EDGE_KERNEL_CACHE:
        _EDGE_KERNEL_CACHE[epad] = _make_edge_kernel(epad)
    num, dpk = _EDGE_KERNEL_CACHE[epad](kv, q, src, dst)
    return num, dpk.reshape(NC, NPAD, H)


def kernel(x_functions, x_classes, x_code, ei_ff, ei_cf, ei_fc, ei_codef,
           params):
    f32 = jnp.float32
    e = ei_ff.shape[1]
    epad = -(-e // (NW * C)) * (NW * C)

    def padn(x):
        return jnp.concatenate([x, jnp.zeros((NPAD - N, D), f32)], axis=0)

    xs = {'functions': padn(x_functions), 'classes': padn(x_classes),
          'code': padn(x_code)}

    srcs, dsts = [], []
    for ei in (ei_ff, ei_cf, ei_fc, ei_codef):
        srcs.append(jnp.concatenate(
            [ei[0], jnp.zeros((epad - e,), jnp.int32)]))
        dsts.append(jnp.concatenate(
            [ei[1], jnp.full((epad - e,), N, jnp.int32)]))

    eye = jnp.eye(H, dtype=f32)

    def bd(w3):
        return (w3[:, :, None, :] * eye[:, None, :, None]).reshape(D, D)

    lhs, rhs = [], []
    for l in range(L):
        for i, (s, _) in enumerate(_ET):
            lhs.append(params['l%d_Wk_%s' % (l, s)])
            rhs.append(bd(params['l%d_Watt_%d' % (l, i)]) * 0.25)
            lhs.append(params['l%d_Wv_%s' % (l, s)])
            rhs.append(bd(params['l%d_Wmsg_%d' % (l, i)]))
    A = _bmm(jnp.stack(lhs), jnp.stack(rhs))

    ilv = (jnp.arange(D) % H) * DH + jnp.arange(D) // H
    A = A[:, :, ilv]

    for l in range(L):
        b = l * len(_ET)
        wf = jnp.concatenate([A[(b + 0) * 2], A[(b + 0) * 2 + 1],
                              A[(b + 2) * 2], A[(b + 2) * 2 + 1],
                              params['l%d_Wq_functions' % l][:, ilv]], axis=1)
        wc = jnp.concatenate([A[(b + 1) * 2], A[(b + 1) * 2 + 1],
                              params['l%d_Wq_classes' % l][:, ilv]], axis=1)
        wcode = jnp.concatenate([A[(b + 3) * 2], A[(b + 3) * 2 + 1]], axis=1)
        yf = _mm(xs['functions'], wf)
        yc = _mm(xs['classes'], wc)
        ycode = _mm(xs['code'], wcode)
        kv_ff, kv_fc, qf = yf[:, 0:256], yf[:, 256:512], yf[:, 512:640]
        kv_cf, qc = yc[:, 0:256], yc[:, 256:384]

        p_ff, d_ff = _edge(kv_ff, qf, srcs[0], dsts[0])
        p_cf, d_cf = _edge(kv_cf, qf, srcs[1], dsts[1])
        p_fc, d_fc = _edge(kv_fc, qc, srcs[2], dsts[2])
        p_codef, d_codef = _edge(ycode, qf, srcs[3], dsts[3])

        pf = jnp.concatenate([p_ff, p_cf, p_codef], axis=0)
        df = jnp.concatenate([d_ff, d_cf, d_codef], axis=0)
        xs = {'functions': _combine(pf, df, xs['functions'],
                                    params['l%d_Wa_functions' % l][ilv, :]),
              'classes': _combine(p_fc, d_fc, xs['classes'],
                                  params['l%d_Wa_classes' % l][ilv, :]),
              'code': xs['code']}

    xf = xs['functions'][:N]
    xc = xs['classes'][:N]
    xcat = jnp.concatenate([xf[:N - 1], xc], axis=0)
    mrows = 2 * N - 1
    mpad = -(-mrows // 1024) * 1024
    x_in = jnp.concatenate(
        [xcat, jnp.zeros((mpad - mrows, D), f32)], axis=0)
    t = xf[N - 1:N]
    w1 = params['mlp_W1']
    w2p = jnp.concatenate(
        [params['mlp_W2'], jnp.zeros((D, D - 1), f32)], axis=1)
    b2 = jnp.broadcast_to(params['mlp_b2'].reshape(1, 1), (1, D))
    scores = _mlp(x_in, t, w1[:D], w1[D:], params['mlp_b1'].reshape(1, D),
                  w2p, b2)
    return scores[:mrows, 0:1]


# speedup vs baseline: 1.1160x; 1.1160x over previous
import functools

import jax
import jax.numpy as jnp
from jax import lax
from jax.experimental import pallas as pl
from jax.experimental.pallas import tpu as pltpu
from jax.experimental.pallas import tpu_sc as plsc

N = 10000
D = 128
H = 8
DH = 16
L = 2
NPAD = 10240
NC, NS, LANES = 2, 16, 16
NW = NC * NS
C = 32
AW = 144
RPT = NPAD // NS

_ET = [('functions', 'functions'), ('classes', 'functions'),
       ('functions', 'classes'), ('code', 'functions')]


def _mm_body(x_ref, w_ref, o_ref):
    o_ref[...] = jnp.dot(x_ref[...], w_ref[...],
                         preferred_element_type=jnp.float32)


def _mm(x, w, bm=1024):
    m, k = x.shape
    n = w.shape[1]
    return pl.pallas_call(
        _mm_body,
        grid=(m // bm,),
        in_specs=[pl.BlockSpec((bm, k), lambda i: (i, 0)),
                  pl.BlockSpec((k, n), lambda i: (0, 0))],
        out_specs=pl.BlockSpec((bm, n), lambda i: (i, 0)),
        out_shape=jax.ShapeDtypeStruct((m, n), jnp.float32),
    )(x, w)


def _bmm_body(a_ref, b_ref, o_ref):
    o_ref[0] = jnp.dot(a_ref[0], b_ref[0], preferred_element_type=jnp.float32)


def _bmm(a, b):
    bsz = a.shape[0]
    return pl.pallas_call(
        _bmm_body,
        grid=(bsz,),
        in_specs=[pl.BlockSpec((1, D, D), lambda i: (i, 0, 0)),
                  pl.BlockSpec((1, D, D), lambda i: (i, 0, 0))],
        out_specs=pl.BlockSpec((1, D, D), lambda i: (i, 0, 0)),
        out_shape=jax.ShapeDtypeStruct((bsz, D, D), jnp.float32),
    )(a, b)


def _combine_body(n_et, s_ref, d_ref, x_ref, wa_ref, o_ref):
    m = None
    for i in range(n_et):
        num = s_ref[2 * i] + s_ref[2 * i + 1]
        den = d_ref[2 * i] + d_ref[2 * i + 1]
        den_w = jnp.concatenate([den] * DH, axis=1)
        agg = num / (den_w + 1e-9)
        m = agg if m is None else jnp.maximum(m, agg)
    g = jax.nn.gelu(m)
    o_ref[...] = jnp.maximum(
        jnp.dot(g, wa_ref[...], preferred_element_type=jnp.float32)
        + x_ref[...], 0.0)


def _combine(parts, dens, x, wa, bm=1024):
    ne2 = parts.shape[0]
    return pl.pallas_call(
        functools.partial(_combine_body, ne2 // 2),
        grid=(NPAD // bm,),
        in_specs=[pl.BlockSpec((ne2, bm, D), lambda i: (0, i, 0)),
                  pl.BlockSpec((ne2, bm, H), lambda i: (0, i, 0)),
                  pl.BlockSpec((bm, D), lambda i: (i, 0)),
                  pl.BlockSpec((D, D), lambda i: (0, 0))],
        out_specs=pl.BlockSpec((bm, D), lambda i: (i, 0)),
        out_shape=jax.ShapeDtypeStruct((NPAD, D), jnp.float32),
    )(parts, dens, x, wa)


def _mlp_body(x_ref, t_ref, w1a_ref, w1b_ref, b1_ref, w2_ref, b2_ref, o_ref):
    xb = x_ref[...]
    xn = xb / (jnp.sqrt(jnp.sum(xb * xb, axis=1, keepdims=True)) + 1e-12)
    t = t_ref[...]
    tn = t / (jnp.sqrt(jnp.sum(t * t)) + 1e-12)
    bias = b1_ref[...] + jnp.dot(tn, w1b_ref[...],
                                 preferred_element_type=jnp.float32)
    hh = jnp.maximum(
        jnp.dot(xn, w1a_ref[...], preferred_element_type=jnp.float32) + bias,
        0.0)
    o_ref[...] = jnp.dot(hh, w2_ref[...],
                         preferred_element_type=jnp.float32) + b2_ref[...]


def _mlp(x, t, w1a, w1b, b1, w2, b2, bm=1024):
    m = x.shape[0]
    return pl.pallas_call(
        _mlp_body,
        grid=(m // bm,),
        in_specs=[pl.BlockSpec((bm, D), lambda i: (i, 0)),
                  pl.BlockSpec((1, D), lambda i: (0, 0)),
                  pl.BlockSpec((D, D), lambda i: (0, 0)),
                  pl.BlockSpec((D, D), lambda i: (0, 0)),
                  pl.BlockSpec((1, D), lambda i: (0, 0)),
                  pl.BlockSpec((D, D), lambda i: (0, 0)),
                  pl.BlockSpec((1, D), lambda i: (0, 0))],
        out_specs=pl.BlockSpec((bm, D), lambda i: (i, 0)),
        out_shape=jax.ShapeDtypeStruct((m, D), jnp.float32),
    )(x, t, w1a, w1b, b1, w2, b2)


NPD = NPAD // LANES


def _perm(v, idx):
    return lax.gather(v, idx[:, None],
                      lax.GatherDimensionNumbers((), (0,), (0,)), (1,),
                      mode=lax.GatherScatterMode.PROMISE_IN_BOUNDS)


def _make_edge_kernel(epad, interpret=False):
    ew = epad // NW
    chunks = ew // C
    mesh = plsc.VectorSubcoreMesh(core_axis_name="c", subcore_axis_name="s",
                                  num_cores=NC, num_subcores=NS)

    @functools.partial(
        pl.kernel,
        out_type=(jax.ShapeDtypeStruct((NC, NPAD, D), jnp.float32),
                  jax.ShapeDtypeStruct((NC, NPD, D), jnp.float32)),
        mesh=mesh,
        interpret=interpret,
        compiler_params=pltpu.CompilerParams(needs_layout_passes=False),
        scratch_types=[
            pltpu.VMEM((C,), jnp.int32),
            pltpu.VMEM((C,), jnp.int32),
            pltpu.VMEM((C,), jnp.int32),
            pltpu.VMEM((C,), jnp.int32),
            pltpu.VMEM((C,), jnp.int32),
            pltpu.VMEM((C,), jnp.int32),
            pltpu.VMEM((C,), jnp.int32),
            pltpu.VMEM((C,), jnp.int32),
            pltpu.VMEM((C, 2 * D), jnp.float32),
            pltpu.VMEM((C, 2 * D), jnp.float32),
            pltpu.VMEM((C, D), jnp.float32),
            pltpu.VMEM((C, D), jnp.float32),
            pltpu.VMEM((C, D), jnp.float32),
            pltpu.VMEM((C, D), jnp.float32),
            pltpu.VMEM((C, LANES), jnp.float32),
            pltpu.VMEM((C, D), jnp.float32),
            pltpu.VMEM_SHARED((NPAD, D), jnp.float32),
            pltpu.VMEM_SHARED((NPD, D), jnp.float32),
            pltpu.SemaphoreType.DMA,
            pltpu.SemaphoreType.DMA,
            pltpu.SemaphoreType.DMA,
            pltpu.SemaphoreType.DMA,
            pltpu.SemaphoreType.DMA,
            pltpu.SemaphoreType.DMA,
            pltpu.SemaphoreType.DMA,
        ],
    )
    def edge_kernel(kv_hbm, q_hbm, src_hbm, dst_hbm, out_hbm, outd_hbm,
                    src0, src1, dst0, dst1, dsc0, dsc1, dhi, pcb,
                    kv0, kv1, q0, q1,
                    ob0, ob1, exbuf, dbuf, acc, accd,
                    gsem0, gsem1, isem0, isem1, ssem0, ssem1, dsem):
        cid = lax.axis_index("c")
        sid = lax.axis_index("s")
        wid = sid * NC + cid
        iot = lax.iota(jnp.int32, LANES)
        zeros16 = jnp.zeros((LANES,), jnp.float32)
        kvb = (kv0, kv1)
        qb = (q0, q1)
        srcb = (src0, src1)
        dstb = (dst0, dst1)
        gsem = (gsem0, gsem1)
        isem = (isem0, isem1)

        def ifetch(g, b):
            eb = pl.multiple_of(wid * ew + g * C, C)
            pltpu.async_copy(src_hbm.at[pl.ds(eb, C)], srcb[b], isem[b])
            pltpu.async_copy(dst_hbm.at[pl.ds(eb, C)], dstb[b], isem[b])

        def iwait(b):
            pltpu.make_async_copy(src_hbm.at[pl.ds(0, C)], srcb[b],
                                  isem[b]).wait()
            pltpu.make_async_copy(dst_hbm.at[pl.ds(0, C)], dstb[b],
                                  isem[b]).wait()

        def zrow(r, carry):
            for jj in range(D // LANES):
                ob0[r, pl.ds(jj * LANES, LANES)] = zeros16
                dbuf[r, pl.ds(jj * LANES, LANES)] = zeros16
            return carry
        lax.fori_loop(0, C, zrow, 0)
        base_r = sid * RPT
        for j in range(RPT // C):
            pltpu.sync_copy(ob0, acc.at[pl.ds(base_r + j * C, C)])
        rd = NPD // NS
        off = 0
        while off < rd:
            step = min(C, rd - off)
            pltpu.sync_copy(dbuf.at[pl.ds(0, step)],
                            accd.at[pl.ds(sid * rd + off, step)])
            off += step
        plsc.subcore_barrier()

        def gissue(b):
            pltpu.async_copy(kv_hbm.at[srcb[b]], kvb[b], gsem[b])
            pltpu.async_copy(q_hbm.at[dstb[b]], qb[b], gsem[b])

        def gwait(b):
            pltpu.make_async_copy(kv_hbm.at[pl.ds(0, C)], kvb[b],
                                  gsem[b]).wait()
            pltpu.make_async_copy(q_hbm.at[pl.ds(0, C)], qb[b],
                                  gsem[b]).wait()

        ifetch(0, 0)
        iwait(0)
        gissue(0)
        ifetch(1, 1)

        obb = (ob0, ob1)
        dscb = (dsc0, dsc1)
        ssem = (ssem0, ssem1)
        rot8 = iot ^ 8
        low8 = iot & 7

        def swait(b):
            pltpu.make_async_copy(out_hbm.at[0, pl.ds(0, C)], obb[b],
                                  ssem[b]).wait()

        def pair(p, carry):
            for b in range(2):
                g = p * 2 + b

                gwait(b)

                @pl.when(g + 1 < chunks)
                def _issue_next():
                    iwait(1 - b)
                    gissue(1 - b)

                @pl.when(g >= 2)
                def _drain_prev_scatter():
                    swait(b)

                kvbuf = kvb[b]
                qbuf = qb[b]
                obuf = obb[b]

                def edge(e, ecarry):
                    t = [qbuf[e, pl.ds(LANES * j, LANES)]
                         * kvbuf[e, pl.ds(LANES * j, LANES)]
                         for j in range(H)]
                    while len(t) > 1:
                        t = [t[2 * j] + t[2 * j + 1]
                             for j in range(len(t) // 2)]
                    sv = t[0] + _perm(t[0], rot8)
                    ev = jnp.exp(sv)
                    exbuf[e, pl.ds(0, LANES)] = ev
                    ehv = _perm(ev, low8)
                    for j in range(H):
                        obuf[e, pl.ds(LANES * j, LANES)] = (
                            kvbuf[e, pl.ds(D + LANES * j, LANES)] * ehv)
                    return ecarry
                lax.fori_loop(0, C, edge, 0, unroll=4)

                @pl.when(g >= 1)
                def _drain_den():
                    pltpu.make_async_copy(out_hbm.at[0, pl.ds(0, C)], dbuf,
                                          dsem).wait()
                    for j2 in range(C // LANES):
                        r16 = j2 * LANES + iot
                        pc = pcb[pl.ds(j2 * LANES, LANES)]
                        for h in range(H):
                            plsc.store_scatter(dbuf, [r16, pc + h], zeros16)

                for j2 in range(C // LANES):
                    r16 = j2 * LANES + iot
                    dst16 = dstb[b][pl.ds(j2 * LANES, LANES)]
                    dscb[b][pl.ds(j2 * LANES, LANES)] = dst16
                    dhi[pl.ds(j2 * LANES, LANES)] = (
                        lax.shift_right_logical(dst16, 4))
                    pcol = (dst16 & (LANES - 1)) * H
                    pcb[pl.ds(j2 * LANES, LANES)] = pcol
                    for h in range(H):
                        exh = plsc.load_gather(
                            exbuf, [r16, jnp.full((LANES,), h, jnp.int32)])
                        plsc.store_scatter(dbuf, [r16, pcol + h], exh)

                pltpu.async_copy(obuf, acc.at[dscb[b]], ssem[b], add=True)
                pltpu.async_copy(dbuf, accd.at[dhi], dsem, add=True)

                @pl.when(g + 2 < chunks)
                def _fetch_ahead():
                    ifetch(g + 2, b)
            return carry
        lax.fori_loop(0, chunks // 2, pair, 0)
        swait(0)
        swait(1)
        pltpu.make_async_copy(out_hbm.at[0, pl.ds(0, C)], dbuf, dsem).wait()
        plsc.subcore_barrier()
        for j in range(RPT // C):
            r0 = base_r + j * C
            pltpu.sync_copy(acc.at[pl.ds(r0, C)], out_hbm.at[cid, pl.ds(r0, C)])
        pltpu.sync_copy(accd.at[pl.ds(sid * rd, rd)],
                        outd_hbm.at[cid, pl.ds(sid * rd, rd)])

    return edge_kernel


_EDGE_KERNEL_CACHE = {}


def _edge(kv, q, src, dst):
    epad = src.shape[0]
    if epad not in _EDGE_KERNEL_CACHE:
        _EDGE_KERNEL_CACHE[epad] = _make_edge_kernel(epad)
    num, dpk = _EDGE_KERNEL_CACHE[epad](kv, q, src, dst)
    return num, dpk.reshape(NC, NPAD, H)


def kernel(x_functions, x_classes, x_code, ei_ff, ei_cf, ei_fc, ei_codef,
           params):
    f32 = jnp.float32
    e = ei_ff.shape[1]
    epad = -(-e // (NW * C)) * (NW * C)

    def padn(x):
        return jnp.concatenate([x, jnp.zeros((NPAD - N, D), f32)], axis=0)

    xs = {'functions': padn(x_functions), 'classes': padn(x_classes),
          'code': padn(x_code)}

    srcs, dsts = [], []
    for ei in (ei_ff, ei_cf, ei_fc, ei_codef):
        srcs.append(jnp.concatenate(
            [ei[0], jnp.zeros((epad - e,), jnp.int32)]))
        dsts.append(jnp.concatenate(
            [ei[1], jnp.full((epad - e,), N, jnp.int32)]))

    eye = jnp.eye(H, dtype=f32)

    def bd(w3):
        return (w3[:, :, None, :] * eye[:, None, :, None]).reshape(D, D)

    lhs, rhs = [], []
    for l in range(L):
        for i, (s, _) in enumerate(_ET):
            lhs.append(params['l%d_Wk_%s' % (l, s)])
            rhs.append(bd(params['l%d_Watt_%d' % (l, i)]) * 0.25)
            lhs.append(params['l%d_Wv_%s' % (l, s)])
            rhs.append(bd(params['l%d_Wmsg_%d' % (l, i)]))
    A = _bmm(jnp.stack(lhs), jnp.stack(rhs))

    ilv = (jnp.arange(D) % H) * DH + jnp.arange(D) // H
    A = A[:, :, ilv]

    for l in range(L):
        b = l * len(_ET)
        wf = jnp.concatenate([A[(b + 0) * 2], A[(b + 0) * 2 + 1],
                              A[(b + 2) * 2], A[(b + 2) * 2 + 1],
                              params['l%d_Wq_functions' % l][:, ilv]], axis=1)
        wc = jnp.concatenate([A[(b + 1) * 2], A[(b + 1) * 2 + 1],
                              params['l%d_Wq_classes' % l][:, ilv]], axis=1)
        wcode = jnp.concatenate([A[(b + 3) * 2], A[(b + 3) * 2 + 1]], axis=1)
        yf = _mm(xs['functions'], wf)
        yc = _mm(xs['classes'], wc)
        ycode = _mm(xs['code'], wcode)
        kv_ff, kv_fc, qf = yf[:, 0:256], yf[:, 256:512], yf[:, 512:640]
        kv_cf, qc = yc[:, 0:256], yc[:, 256:384]

        p_ff, d_ff = _edge(kv_ff, qf, srcs[0], dsts[0])
        p_cf, d_cf = _edge(kv_cf, qf, srcs[1], dsts[1])
        p_fc, d_fc = _edge(kv_fc, qc, srcs[2], dsts[2])
        p_codef, d_codef = _edge(ycode, qf, srcs[3], dsts[3])

        pf = jnp.concatenate([p_ff, p_cf, p_codef], axis=0)
        df = jnp.concatenate([d_ff, d_cf, d_codef], axis=0)
        xs = {'functions': _combine(pf, df, xs['functions'],
                                    params['l%d_Wa_functions' % l][ilv, :]),
              'classes': _combine(p_fc, d_fc, xs['classes'],
                                  params['l%d_Wa_classes' % l][ilv, :]),
              'code': xs['code']}

    xf = xs['functions'][:N]
    xc = xs['classes'][:N]
    xcat = jnp.concatenate([xf[:N - 1], xc], axis=0)
    mrows = 2 * N - 1
    mpad = -(-mrows // 1024) * 1024
    x_in = jnp.concatenate(
        [xcat, jnp.zeros((mpad - mrows, D), f32)], axis=0)
    t = xf[N - 1:N]
    w1 = params['mlp_W1']
    w2p = jnp.concatenate(
        [params['mlp_W2'], jnp.zeros((D, D - 1), f32)], axis=1)
    b2 = jnp.broadcast_to(params['mlp_b2'].reshape(1, 1), (1, D))
    scores = _mlp(x_in, t, w1[:D], w1[D:], params['mlp_b1'].reshape(1, D),
                  w2p, b2)
    return scores[:mrows, 0:1]


# speedup vs baseline: 1.1569x; 1.0366x over previous
import functools

import jax
import jax.numpy as jnp
from jax import lax
from jax.experimental import pallas as pl
from jax.experimental.pallas import tpu as pltpu
from jax.experimental.pallas import tpu_sc as plsc

N = 10000
D = 128
H = 8
DH = 16
L = 2
NPAD = 10240
NC, NS, LANES = 2, 16, 16
NW = NC * NS
C = 32
RPT = NPAD // NS

_ET = [('functions', 'functions'), ('classes', 'functions'),
       ('functions', 'classes'), ('code', 'functions')]


def _mm_body(x_ref, w_ref, o_ref):
    o_ref[...] = jnp.dot(x_ref[...], w_ref[...],
                         preferred_element_type=jnp.float32)


def _mm(x, w, bm=1024):
    m, k = x.shape
    n = w.shape[1]
    return pl.pallas_call(
        _mm_body,
        grid=(m // bm,),
        in_specs=[pl.BlockSpec((bm, k), lambda i: (i, 0)),
                  pl.BlockSpec((k, n), lambda i: (0, 0))],
        out_specs=pl.BlockSpec((bm, n), lambda i: (i, 0)),
        out_shape=jax.ShapeDtypeStruct((m, n), jnp.float32),
    )(x, w)


def _bmm_body(a_ref, b_ref, o_ref):
    o_ref[0] = jnp.dot(a_ref[0], b_ref[0], preferred_element_type=jnp.float32)


def _bmm(a, b):
    bsz = a.shape[0]
    return pl.pallas_call(
        _bmm_body,
        grid=(bsz,),
        in_specs=[pl.BlockSpec((1, D, D), lambda i: (i, 0, 0)),
                  pl.BlockSpec((1, D, D), lambda i: (i, 0, 0))],
        out_specs=pl.BlockSpec((1, D, D), lambda i: (i, 0, 0)),
        out_shape=jax.ShapeDtypeStruct((bsz, D, D), jnp.float32),
    )(a, b)


def _combine_body(n_et, s_ref, d_ref, x_ref, wa_ref, o_ref):
    m = None
    for i in range(n_et):
        num = s_ref[2 * i] + s_ref[2 * i + 1]
        den = d_ref[2 * i] + d_ref[2 * i + 1]
        den_w = jnp.concatenate([den] * DH, axis=1)
        agg = num / (den_w + 1e-9)
        m = agg if m is None else jnp.maximum(m, agg)
    g = jax.nn.gelu(m)
    o_ref[...] = jnp.maximum(
        jnp.dot(g, wa_ref[...], preferred_element_type=jnp.float32)
        + x_ref[...], 0.0)


def _combine(parts, dens, x, wa, bm=1024):
    ne2 = parts.shape[0]
    return pl.pallas_call(
        functools.partial(_combine_body, ne2 // 2),
        grid=(NPAD // bm,),
        in_specs=[pl.BlockSpec((ne2, bm, D), lambda i: (0, i, 0)),
                  pl.BlockSpec((ne2, bm, H), lambda i: (0, i, 0)),
                  pl.BlockSpec((bm, D), lambda i: (i, 0)),
                  pl.BlockSpec((D, D), lambda i: (0, 0))],
        out_specs=pl.BlockSpec((bm, D), lambda i: (i, 0)),
        out_shape=jax.ShapeDtypeStruct((NPAD, D), jnp.float32),
    )(parts, dens, x, wa)


def _mlp_body(x_ref, t_ref, w1a_ref, w1b_ref, b1_ref, w2_ref, b2_ref, o_ref):
    xb = x_ref[...]
    xn = xb / (jnp.sqrt(jnp.sum(xb * xb, axis=1, keepdims=True)) + 1e-12)
    t = t_ref[...]
    tn = t / (jnp.sqrt(jnp.sum(t * t)) + 1e-12)
    bias = b1_ref[...] + jnp.dot(tn, w1b_ref[...],
                                 preferred_element_type=jnp.float32)
    hh = jnp.maximum(
        jnp.dot(xn, w1a_ref[...], preferred_element_type=jnp.float32) + bias,
        0.0)
    o_ref[...] = jnp.dot(hh, w2_ref[...],
                         preferred_element_type=jnp.float32) + b2_ref[...]


def _mlp(x, t, w1a, w1b, b1, w2, b2, bm=1024):
    m = x.shape[0]
    return pl.pallas_call(
        _mlp_body,
        grid=(m // bm,),
        in_specs=[pl.BlockSpec((bm, D), lambda i: (i, 0)),
                  pl.BlockSpec((1, D), lambda i: (0, 0)),
                  pl.BlockSpec((D, D), lambda i: (0, 0)),
                  pl.BlockSpec((D, D), lambda i: (0, 0)),
                  pl.BlockSpec((1, D), lambda i: (0, 0)),
                  pl.BlockSpec((D, D), lambda i: (0, 0)),
                  pl.BlockSpec((1, D), lambda i: (0, 0))],
        out_specs=pl.BlockSpec((bm, D), lambda i: (i, 0)),
        out_shape=jax.ShapeDtypeStruct((m, D), jnp.float32),
    )(x, t, w1a, w1b, b1, w2, b2)


NPD = NPAD // LANES


def _perm(v, idx):
    return lax.gather(v, idx[:, None],
                      lax.GatherDimensionNumbers((), (0,), (0,)), (1,),
                      mode=lax.GatherScatterMode.PROMISE_IN_BOUNDS)


def _make_edge_kernel(epad, interpret=False):
    ew = epad // NW
    chunks = ew // C
    mesh = plsc.VectorSubcoreMesh(core_axis_name="c", subcore_axis_name="s",
                                  num_cores=NC, num_subcores=NS)

    @functools.partial(
        pl.kernel,
        out_type=(jax.ShapeDtypeStruct((NC, NPAD, D), jnp.float32),
                  jax.ShapeDtypeStruct((NC, NPD, D), jnp.float32)),
        mesh=mesh,
        interpret=interpret,
        compiler_params=pltpu.CompilerParams(needs_layout_passes=False),
        scratch_types=[
            pltpu.VMEM((C,), jnp.int32),
            pltpu.VMEM((C,), jnp.int32),
            pltpu.VMEM((C,), jnp.int32),
            pltpu.VMEM((C,), jnp.int32),
            pltpu.VMEM((C,), jnp.int32),
            pltpu.VMEM((C,), jnp.int32),
            pltpu.VMEM((C,), jnp.int32),
            pltpu.VMEM((C,), jnp.int32),
            pltpu.VMEM((C, 2 * D), jnp.float32),
            pltpu.VMEM((C, 2 * D), jnp.float32),
            pltpu.VMEM((C, D), jnp.float32),
            pltpu.VMEM((C, D), jnp.float32),
            pltpu.VMEM((C, D), jnp.float32),
            pltpu.VMEM((C, D), jnp.float32),
            pltpu.VMEM((C, LANES), jnp.float32),
            pltpu.VMEM((C, D), jnp.float32),
            pltpu.VMEM_SHARED((NPAD, D), jnp.float32),
            pltpu.VMEM_SHARED((NPD, D), jnp.float32),
            pltpu.SemaphoreType.DMA,
            pltpu.SemaphoreType.DMA,
            pltpu.SemaphoreType.DMA,
            pltpu.SemaphoreType.DMA,
            pltpu.SemaphoreType.DMA,
            pltpu.SemaphoreType.DMA,
            pltpu.SemaphoreType.DMA,
        ],
    )
    def edge_kernel(kv_hbm, q_hbm, src_hbm, dst_hbm, out_hbm, outd_hbm,
                    src0, src1, dst0, dst1, dsc0, dsc1, dhi, pcb,
                    kv0, kv1, q0, q1,
                    ob0, ob1, exbuf, dbuf, acc, accd,
                    gsem0, gsem1, isem0, isem1, ssem0, ssem1, dsem):
        cid = lax.axis_index("c")
        sid = lax.axis_index("s")
        wid = sid * NC + cid
        iot = lax.iota(jnp.int32, LANES)
        zeros16 = jnp.zeros((LANES,), jnp.float32)
        kvb = (kv0, kv1)
        qb = (q0, q1)
        srcb = (src0, src1)
        dstb = (dst0, dst1)
        gsem = (gsem0, gsem1)
        isem = (isem0, isem1)

        def ifetch(g, b):
            eb = pl.multiple_of(wid * ew + g * C, C)
            pltpu.async_copy(src_hbm.at[pl.ds(eb, C)], srcb[b], isem[b])
            pltpu.async_copy(dst_hbm.at[pl.ds(eb, C)], dstb[b], isem[b])

        def iwait(b):
            pltpu.make_async_copy(src_hbm.at[pl.ds(0, C)], srcb[b],
                                  isem[b]).wait()
            pltpu.make_async_copy(dst_hbm.at[pl.ds(0, C)], dstb[b],
                                  isem[b]).wait()

        def zrow(r, carry):
            for jj in range(D // LANES):
                ob0[r, pl.ds(jj * LANES, LANES)] = zeros16
                dbuf[r, pl.ds(jj * LANES, LANES)] = zeros16
            return carry
        lax.fori_loop(0, C, zrow, 0)
        base_r = sid * RPT
        for j in range(RPT // C):
            pltpu.sync_copy(ob0, acc.at[pl.ds(base_r + j * C, C)])
        rd = NPD // NS
        off = 0
        while off < rd:
            step = min(C, rd - off)
            pltpu.sync_copy(dbuf.at[pl.ds(0, step)],
                            accd.at[pl.ds(sid * rd + off, step)])
            off += step
        plsc.subcore_barrier()

        def gissue(b):
            pltpu.async_copy(kv_hbm.at[srcb[b]], kvb[b], gsem[b])
            pltpu.async_copy(q_hbm.at[dstb[b]], qb[b], gsem[b])

        def gwait(b):
            pltpu.make_async_copy(kv_hbm.at[pl.ds(0, C)], kvb[b],
                                  gsem[b]).wait()
            pltpu.make_async_copy(q_hbm.at[pl.ds(0, C)], qb[b],
                                  gsem[b]).wait()

        ifetch(0, 0)
        iwait(0)
        gissue(0)
        ifetch(1, 1)

        obb = (ob0, ob1)
        dscb = (dsc0, dsc1)
        ssem = (ssem0, ssem1)
        rot8 = iot ^ 8
        low8 = iot & 7

        def swait(b):
            pltpu.make_async_copy(out_hbm.at[0, pl.ds(0, C)], obb[b],
                                  ssem[b]).wait()

        def pair(p, carry):
            for b in range(2):
                g = p * 2 + b

                gwait(b)

                @pl.when(g + 1 < chunks)
                def _issue_next():
                    iwait(1 - b)
                    gissue(1 - b)

                @pl.when(g >= 2)
                def _drain_prev_scatter():
                    swait(b)

                kvbuf = kvb[b]
                qbuf = qb[b]
                obuf = obb[b]

                def edge(e, ecarry):
                    t = [qbuf[e, pl.ds(LANES * j, LANES)]
                         * kvbuf[e, pl.ds(LANES * j, LANES)]
                         for j in range(H)]
                    while len(t) > 1:
                        t = [t[2 * j] + t[2 * j + 1]
                             for j in range(len(t) // 2)]
                    sv = t[0] + _perm(t[0], rot8)
                    ev = jnp.exp(sv)
                    exbuf[e, pl.ds(0, LANES)] = ev
                    ehv = _perm(ev, low8)
                    for j in range(H):
                        obuf[e, pl.ds(LANES * j, LANES)] = (
                            kvbuf[e, pl.ds(D + LANES * j, LANES)] * ehv)
                    return ecarry
                lax.fori_loop(0, C, edge, 0, unroll=8)

                @pl.when(g >= 1)
                def _drain_den():
                    pltpu.make_async_copy(out_hbm.at[0, pl.ds(0, C)], dbuf,
                                          dsem).wait()
                    for j2 in range(C // LANES):
                        r16 = j2 * LANES + iot
                        pc = pcb[pl.ds(j2 * LANES, LANES)]
                        for h in range(H):
                            plsc.store_scatter(dbuf, [r16, pc + h], zeros16)

                for j2 in range(C // LANES):
                    r16 = j2 * LANES + iot
                    dst16 = dstb[b][pl.ds(j2 * LANES, LANES)]
                    dscb[b][pl.ds(j2 * LANES, LANES)] = dst16
                    dhi[pl.ds(j2 * LANES, LANES)] = (
                        lax.shift_right_logical(dst16, 4))
                    pcol = (dst16 & (LANES - 1)) * H
                    pcb[pl.ds(j2 * LANES, LANES)] = pcol
                    for h in range(H):
                        exh = plsc.load_gather(
                            exbuf, [r16, jnp.full((LANES,), h, jnp.int32)])
                        plsc.store_scatter(dbuf, [r16, pcol + h], exh)

                pltpu.async_copy(obuf, acc.at[dscb[b]], ssem[b], add=True)
                pltpu.async_copy(dbuf, accd.at[dhi], dsem, add=True)

                @pl.when(g + 2 < chunks)
                def _fetch_ahead():
                    ifetch(g + 2, b)
            return carry
        lax.fori_loop(0, chunks // 2, pair, 0)
        swait(0)
        swait(1)
        pltpu.make_async_copy(out_hbm.at[0, pl.ds(0, C)], dbuf, dsem).wait()
        plsc.subcore_barrier()
        for j in range(RPT // C):
            r0 = base_r + j * C
            pltpu.sync_copy(acc.at[pl.ds(r0, C)], out_hbm.at[cid, pl.ds(r0, C)])
        pltpu.sync_copy(accd.at[pl.ds(sid * rd, rd)],
                        outd_hbm.at[cid, pl.ds(sid * rd, rd)])

    return edge_kernel


_EDGE_KERNEL_CACHE = {}


def _edge(kv, q, src, dst):
    epad = src.shape[0]
    if epad not in _EDGE_KERNEL_CACHE:
        _EDGE_KERNEL_CACHE[epad] = _make_edge_kernel(epad)
    num, dpk = _EDGE_KERNEL_CACHE[epad](kv, q, src, dst)
    return num, dpk.reshape(NC, NPAD, H)


def kernel(x_functions, x_classes, x_code, ei_ff, ei_cf, ei_fc, ei_codef,
           params):
    f32 = jnp.float32
    e = ei_ff.shape[1]
    epad = -(-e // (NW * C)) * (NW * C)

    def padn(x):
        return jnp.concatenate([x, jnp.zeros((NPAD - N, D), f32)], axis=0)

    xs = {'functions': padn(x_functions), 'classes': padn(x_classes),
          'code': padn(x_code)}

    srcs, dsts = [], []
    for ei in (ei_ff, ei_cf, ei_fc, ei_codef):
        srcs.append(jnp.concatenate(
            [ei[0], jnp.zeros((epad - e,), jnp.int32)]))
        dsts.append(jnp.concatenate(
            [ei[1], jnp.full((epad - e,), N, jnp.int32)]))

    eye = jnp.eye(H, dtype=f32)

    def bd(w3):
        return (w3[:, :, None, :] * eye[:, None, :, None]).reshape(D, D)

    lhs, rhs = [], []
    for l in range(L):
        for i, (s, _) in enumerate(_ET):
            lhs.append(params['l%d_Wk_%s' % (l, s)])
            rhs.append(bd(params['l%d_Watt_%d' % (l, i)]) * 0.25)
            lhs.append(params['l%d_Wv_%s' % (l, s)])
            rhs.append(bd(params['l%d_Wmsg_%d' % (l, i)]))
    A = _bmm(jnp.stack(lhs), jnp.stack(rhs))

    ilv = (jnp.arange(D) % H) * DH + jnp.arange(D) // H
    A = A[:, :, ilv]

    for l in range(L):
        b = l * len(_ET)
        wf = jnp.concatenate([A[(b + 0) * 2], A[(b + 0) * 2 + 1],
                              A[(b + 2) * 2], A[(b + 2) * 2 + 1],
                              params['l%d_Wq_functions' % l][:, ilv]], axis=1)
        wc = jnp.concatenate([A[(b + 1) * 2], A[(b + 1) * 2 + 1],
                              params['l%d_Wq_classes' % l][:, ilv]], axis=1)
        wcode = jnp.concatenate([A[(b + 3) * 2], A[(b + 3) * 2 + 1]], axis=1)
        yf = _mm(xs['functions'], wf)
        yc = _mm(xs['classes'], wc)
        ycode = _mm(xs['code'], wcode)
        kv_ff, kv_fc, qf = yf[:, 0:256], yf[:, 256:512], yf[:, 512:640]
        kv_cf, qc = yc[:, 0:256], yc[:, 256:384]

        p_ff, d_ff = _edge(kv_ff, qf, srcs[0], dsts[0])
        p_cf, d_cf = _edge(kv_cf, qf, srcs[1], dsts[1])
        p_fc, d_fc = _edge(kv_fc, qc, srcs[2], dsts[2])
        p_codef, d_codef = _edge(ycode, qf, srcs[3], dsts[3])

        pf = jnp.concatenate([p_ff, p_cf, p_codef], axis=0)
        df = jnp.concatenate([d_ff, d_cf, d_codef], axis=0)
        xs = {'functions': _combine(pf, df, xs['functions'],
                                    params['l%d_Wa_functions' % l][ilv, :]),
              'classes': _combine(p_fc, d_fc, xs['classes'],
                                  params['l%d_Wa_classes' % l][ilv, :]),
              'code': xs['code']}

    xf = xs['functions'][:N]
    xc = xs['classes'][:N]
    xcat = jnp.concatenate([xf[:N - 1], xc], axis=0)
    mrows = 2 * N - 1
    mpad = -(-mrows // 1024) * 1024
    x_in = jnp.concatenate(
        [xcat, jnp.zeros((mpad - mrows, D), f32)], axis=0)
    t = xf[N - 1:N]
    w1 = params['mlp_W1']
    w2p = jnp.concatenate(
        [params['mlp_W2'], jnp.zeros((D, D - 1), f32)], axis=1)
    b2 = jnp.broadcast_to(params['mlp_b2'].reshape(1, 1), (1, D))
    scores = _mlp(x_in, t, w1[:D], w1[D:], params['mlp_b1'].reshape(1, D),
                  w2p, b2)
    return scores[:mrows, 0:1]
